# baseline (speedup 1.0000x reference)
import functools

import jax
import jax.numpy as jnp
from jax.experimental import pallas as pl
from jax.experimental.pallas import tpu as pltpu

C = 64
NB = 8
SEG = 4096
QT = 512
KS = (8, 16, 32)
KMAX = 32
G = 8
GW = SEG // G
RMAX = KMAX + 1
IMAX = 2147483647


def _h_body(x_ref, w_ref, b_ref, g_ref, be_ref, h_ref):
    h = jnp.dot(x_ref[...], w_ref[...], preferred_element_type=jnp.float32)
    h = (h + b_ref[...]) * g_ref[...] + be_ref[...]
    h_ref[...] = jnp.maximum(h, 0.0)


def _sortable(d):
    i = jax.lax.bitcast_convert_type(d, jnp.int32)
    return jnp.where(i >= 0, i, jnp.bitwise_xor(jnp.bitwise_not(i),
                                                -2147483648))


def _knn_body(pt_ref, sq_ref, pq_ref, h_ref, out_ref, keys_ref, wk_ref,
              e_ref, done_ref):
    pt = pt_ref[0]
    sq = sq_ref[0]
    pq = pq_ref[0]
    sq_q = jnp.sum(pq * pq, axis=1, keepdims=True)
    d = sq_q + sq - 2.0 * jnp.dot(pq, pt,
                                  preferred_element_type=jnp.float32)
    keys = _sortable(d)
    keys_ref[...] = keys
    wk_ref[...] = keys
    e_ref[...] = jnp.full((RMAX, QT, G), IMAX, jnp.int32)
    done_ref[0] = 0

    def round_body(r, _):
        @pl.when(done_ref[0] == 0)
        def _():
            ms = []
            for g in range(G):
                sl = wk_ref[:, g * GW:(g + 1) * GW]
                m = jnp.min(sl, axis=1, keepdims=True)
                wk_ref[:, g * GW:(g + 1) * GW] = jnp.where(sl == m, IMAX, sl)
                ms.append(m)
            mrow = jnp.concatenate(ms, axis=1)
            e_ref[r] = mrow
            w = jnp.min(mrow, axis=1)
            cnt = jnp.sum((e_ref[...] < w[None, :, None]).astype(jnp.int32),
                          axis=(0, 2))
            done_ref[0] = jnp.all(cnt >= KMAX).astype(jnp.int32)
        return 0

    jax.lax.fori_loop(0, RMAX, round_body, 0)

    e = e_ref[...]
    thr = {}
    for r in range(KMAX):
        m2 = jnp.min(jnp.min(e, axis=0), axis=1)
        if (r + 1) in KS:
            thr[r + 1] = m2[:, None]
        e = jnp.where(e == m2[None, :, None], IMAX, e)

    hs = h_ref[0]
    for j, k in enumerate(KS):
        mask = (keys_ref[...] <= thr[k]).astype(jnp.float32)
        cnt = jnp.sum(mask, axis=1, keepdims=True)
        s = jnp.dot(mask, hs, preferred_element_type=jnp.float32)
        out_ref[0, :, j * C:(j + 1) * C] = s / cnt


def kernel(p, x, o, W, b, gamma, beta):
    n = p.shape[0]
    h = pl.pallas_call(
        _h_body,
        grid=(n // 2048,),
        in_specs=[
            pl.BlockSpec((2048, C), lambda i: (i, 0)),
            pl.BlockSpec((C, C), lambda i: (0, 0)),
            pl.BlockSpec((1, C), lambda i: (0, 0)),
            pl.BlockSpec((1, C), lambda i: (0, 0)),
            pl.BlockSpec((1, C), lambda i: (0, 0)),
        ],
        out_specs=pl.BlockSpec((2048, C), lambda i: (i, 0)),
        out_shape=jax.ShapeDtypeStruct((n, C), jnp.float32),
    )(x, W, b.reshape(1, C), gamma.reshape(1, C), beta.reshape(1, C))

    p3 = p.reshape(NB, SEG, 3)
    pt = jnp.transpose(p3, (0, 2, 1))
    sq = jnp.sum(p3 * p3, axis=2)[:, None, :]
    h3 = h.reshape(NB, SEG, C)

    knn = pl.pallas_call(
        _knn_body,
        grid=(NB, SEG // QT),
        in_specs=[
            pl.BlockSpec((1, 3, SEG), lambda s, q: (s, 0, 0)),
            pl.BlockSpec((1, 1, SEG), lambda s, q: (s, 0, 0)),
            pl.BlockSpec((1, QT, 3), lambda s, q: (s, q, 0)),
            pl.BlockSpec((1, SEG, C), lambda s, q: (s, 0, 0)),
        ],
        out_specs=pl.BlockSpec((1, QT, 3 * C), lambda s, q: (s, q, 0)),
        out_shape=jax.ShapeDtypeStruct((NB, SEG, 3 * C), jnp.float32),
        scratch_shapes=[
            pltpu.VMEM((QT, SEG), jnp.int32),
            pltpu.VMEM((QT, SEG), jnp.int32),
            pltpu.VMEM((RMAX, QT, G), jnp.int32),
            pltpu.SMEM((1,), jnp.int32),
        ],
        compiler_params=pltpu.CompilerParams(
            dimension_semantics=("arbitrary", "arbitrary"),
        ),
    )(pt, sq, p3, h3)

    out = jnp.concatenate([h, knn.reshape(n, 3 * C)], axis=1)
    return (p, out, o)


# speedup vs baseline: 1.1755x; 1.1755x over previous
import functools

import jax
import jax.numpy as jnp
from jax import lax
from jax.experimental import pallas as pl
from jax.experimental.pallas import tpu as pltpu
from jax.experimental.pallas import tpu_sc as plsc

C = 64
NB = 8
SEG = 4096
QT = 512
KMAX = 32
IMAX = 2147483647

N_TOT = NB * SEG
NW = 32
QPW = N_TOT // NW
BQ = 16
NBLK = QPW // BQ
CP = 128
NIDX = BQ * KMAX
NCH = NIDX // 128


def _h_body(x_ref, w_ref, b_ref, g_ref, be_ref, h_ref):
    h = jnp.dot(x_ref[...], w_ref[...], preferred_element_type=jnp.float32)
    h = (h + b_ref[...]) * g_ref[...] + be_ref[...]
    h_ref[...] = jnp.maximum(h, 0.0)


def _sortable(d):
    i = jax.lax.bitcast_convert_type(d, jnp.int32)
    return jnp.where(i >= 0, i, jnp.bitwise_xor(jnp.bitwise_not(i),
                                                -2147483648))


def _knn_body(pt_ref, sq_ref, pq_ref, idx_ref):
    pt = pt_ref[0]
    sq = sq_ref[0]
    pq = pq_ref[0]
    sq_q = jnp.sum(pq * pq, axis=1, keepdims=True)
    d = sq_q + sq - 2.0 * jnp.dot(pq, pt,
                                  preferred_element_type=jnp.float32)
    wk = _sortable(d)
    iota = lax.broadcasted_iota(jnp.int32, (QT, SEG), 1)
    base = pl.program_id(0) * SEG
    idxs = []
    for _ in range(KMAX):
        m = jnp.min(wk, axis=1, keepdims=True)
        eq = wk == m
        am = jnp.min(jnp.where(eq, iota, IMAX), axis=1, keepdims=True)
        idxs.append(am + base)
        wk = jnp.where(eq, IMAX, wk)
    idx_ref[0] = jnp.concatenate(idxs, axis=1)


def _make_pool():
    mesh = plsc.VectorSubcoreMesh(core_axis_name="c", subcore_axis_name="s")

    @functools.partial(
        pl.kernel, mesh=mesh,
        out_type=jax.ShapeDtypeStruct((N_TOT, 3 * C), jnp.float32),
        scratch_types=[
            pltpu.VMEM((NIDX,), jnp.int32),
            pltpu.VMEM((NIDX, CP), jnp.float32),
            pltpu.VMEM((BQ, 3 * C), jnp.float32),
            pltpu.SemaphoreType.DMA,
        ],
    )
    def pool(idx_hbm, h_hbm, out_hbm, idx_v, rows_v, out_v, sem):
        wid = lax.axis_index("s") * 2 + lax.axis_index("c")
        base = wid * QPW

        def block(blk, carry):
            qb = base + blk * BQ
            pltpu.sync_copy(idx_hbm.at[pl.ds(qb * KMAX, NIDX)], idx_v)
            cps = [
                pltpu.async_copy(
                    h_hbm.at[idx_v.at[pl.ds(j * 128, 128)]],
                    rows_v.at[pl.ds(j * 128, 128), :], sem)
                for j in range(NCH)
            ]
            for cp in cps:
                cp.wait()

            def per_q(q, carry2):
                def accum(lo, hi, acc):
                    def rrow(r, a):
                        return tuple(
                            a[c] + rows_v[q * KMAX + r, pl.ds(c * 16, 16)]
                            for c in range(4))
                    return lax.fori_loop(lo, hi, rrow, acc)

                z = tuple(jnp.zeros((16,), jnp.float32) for _ in range(4))
                a8 = accum(0, 8, z)
                for c in range(4):
                    out_v[q, pl.ds(c * 16, 16)] = a8[c] * 0.125
                a16 = accum(8, 16, a8)
                for c in range(4):
                    out_v[q, pl.ds(C + c * 16, 16)] = a16[c] * 0.0625
                a32 = accum(16, 32, a16)
                for c in range(4):
                    out_v[q, pl.ds(2 * C + c * 16, 16)] = a32[c] * 0.03125
                return carry2

            lax.fori_loop(0, BQ, per_q, 0)
            pltpu.sync_copy(out_v, out_hbm.at[pl.ds(qb, BQ)])
            return carry

        lax.fori_loop(0, NBLK, block, 0)

    return pool


_pool = _make_pool()


def kernel(p, x, o, W, b, gamma, beta):
    n = p.shape[0]
    h = pl.pallas_call(
        _h_body,
        grid=(n // 2048,),
        in_specs=[
            pl.BlockSpec((2048, C), lambda i: (i, 0)),
            pl.BlockSpec((C, C), lambda i: (0, 0)),
            pl.BlockSpec((1, C), lambda i: (0, 0)),
            pl.BlockSpec((1, C), lambda i: (0, 0)),
            pl.BlockSpec((1, C), lambda i: (0, 0)),
        ],
        out_specs=pl.BlockSpec((2048, C), lambda i: (i, 0)),
        out_shape=jax.ShapeDtypeStruct((n, C), jnp.float32),
    )(x, W, b.reshape(1, C), gamma.reshape(1, C), beta.reshape(1, C))

    p3 = p.reshape(NB, SEG, 3)
    pt = jnp.transpose(p3, (0, 2, 1))
    sq = jnp.sum(p3 * p3, axis=2)[:, None, :]

    idx = pl.pallas_call(
        _knn_body,
        grid=(NB, SEG // QT),
        in_specs=[
            pl.BlockSpec((1, 3, SEG), lambda s, q: (s, 0, 0)),
            pl.BlockSpec((1, 1, SEG), lambda s, q: (s, 0, 0)),
            pl.BlockSpec((1, QT, 3), lambda s, q: (s, q, 0)),
        ],
        out_specs=pl.BlockSpec((1, QT, KMAX), lambda s, q: (s, q, 0)),
        out_shape=jax.ShapeDtypeStruct((NB, SEG, KMAX), jnp.int32),
        compiler_params=pltpu.CompilerParams(
            dimension_semantics=("arbitrary", "arbitrary"),
        ),
    )(pt, sq, p3)

    hp = jnp.pad(h, ((0, 0), (0, CP - C)))
    pooled = _pool(idx.reshape(N_TOT * KMAX), hp)
    out = jnp.concatenate([h, pooled], axis=1)
    return (p, out, o)


# speedup vs baseline: 1.4681x; 1.2489x over previous
import functools

import jax
import jax.numpy as jnp
from jax import lax
from jax.experimental import pallas as pl
from jax.experimental.pallas import tpu as pltpu
from jax.experimental.pallas import tpu_sc as plsc

C = 64
NB = 8
SEG = 4096
QT = 512
KMAX = 32
IMAX = 2147483647

N_TOT = NB * SEG
NW = 32
QPW = N_TOT // NW
BQ = 16
NBLK = QPW // BQ
CP = 128
NIDX = BQ * KMAX
NCH = NIDX // 128


def _h_body(x_ref, w_ref, b_ref, g_ref, be_ref, h_ref):
    h = jnp.dot(x_ref[...], w_ref[...], preferred_element_type=jnp.float32)
    h = (h + b_ref[...]) * g_ref[...] + be_ref[...]
    h_ref[...] = jnp.maximum(h, 0.0)


def _sortable(d):
    i = jax.lax.bitcast_convert_type(d, jnp.int32)
    return jnp.where(i >= 0, i, jnp.bitwise_xor(jnp.bitwise_not(i),
                                                -2147483648))


def _knn_body(pt_ref, sq_ref, pq_ref, idx_ref):
    pt = pt_ref[0]
    sq = sq_ref[0]
    pq = pq_ref[0]
    sq_q = jnp.sum(pq * pq, axis=1, keepdims=True)
    d = sq_q + sq - 2.0 * jnp.dot(pq, pt,
                                  preferred_element_type=jnp.float32)
    wk = d
    iota = lax.broadcasted_iota(jnp.int32, (QT, SEG), 1)
    base = pl.program_id(0) * SEG
    idxs = []
    for _ in range(KMAX):
        am = jnp.argmin(wk, axis=1).astype(jnp.int32)[:, None]
        idxs.append(am + base)
        wk = jnp.where(iota == am, jnp.inf, wk)
    idx_ref[0] = jnp.concatenate(idxs, axis=1)


def _make_pool():
    mesh = plsc.VectorSubcoreMesh(core_axis_name="c", subcore_axis_name="s")

    @functools.partial(
        pl.kernel, mesh=mesh,
        out_type=jax.ShapeDtypeStruct((N_TOT, 3 * C), jnp.float32),
        scratch_types=[
            pltpu.VMEM((NIDX,), jnp.int32),
            pltpu.VMEM((NIDX, CP), jnp.float32),
            pltpu.VMEM((BQ, 3 * C), jnp.float32),
            pltpu.SemaphoreType.DMA,
        ],
    )
    def pool(idx_hbm, h_hbm, out_hbm, idx_v, rows_v, out_v, sem):
        wid = lax.axis_index("s") * 2 + lax.axis_index("c")
        base = wid * QPW

        def block(blk, carry):
            qb = base + blk * BQ
            pltpu.sync_copy(idx_hbm.at[pl.ds(qb * KMAX, NIDX)], idx_v)
            cps = [
                pltpu.async_copy(
                    h_hbm.at[idx_v.at[pl.ds(j * 128, 128)]],
                    rows_v.at[pl.ds(j * 128, 128), :], sem)
                for j in range(NCH)
            ]
            for cp in cps:
                cp.wait()

            def per_q(q, carry2):
                def accum(lo, hi, acc):
                    def rrow(r, a):
                        return tuple(
                            a[c] + rows_v[q * KMAX + r, pl.ds(c * 16, 16)]
                            for c in range(4))
                    return lax.fori_loop(lo, hi, rrow, acc)

                z = tuple(jnp.zeros((16,), jnp.float32) for _ in range(4))
                a8 = accum(0, 8, z)
                for c in range(4):
                    out_v[q, pl.ds(c * 16, 16)] = a8[c] * 0.125
                a16 = accum(8, 16, a8)
                for c in range(4):
                    out_v[q, pl.ds(C + c * 16, 16)] = a16[c] * 0.0625
                a32 = accum(16, 32, a16)
                for c in range(4):
                    out_v[q, pl.ds(2 * C + c * 16, 16)] = a32[c] * 0.03125
                return carry2

            lax.fori_loop(0, BQ, per_q, 0)
            pltpu.sync_copy(out_v, out_hbm.at[pl.ds(qb, BQ)])
            return carry

        lax.fori_loop(0, NBLK, block, 0)

    return pool


_pool = _make_pool()


def kernel(p, x, o, W, b, gamma, beta):
    n = p.shape[0]
    h = pl.pallas_call(
        _h_body,
        grid=(n // 2048,),
        in_specs=[
            pl.BlockSpec((2048, C), lambda i: (i, 0)),
            pl.BlockSpec((C, C), lambda i: (0, 0)),
            pl.BlockSpec((1, C), lambda i: (0, 0)),
            pl.BlockSpec((1, C), lambda i: (0, 0)),
            pl.BlockSpec((1, C), lambda i: (0, 0)),
        ],
        out_specs=pl.BlockSpec((2048, C), lambda i: (i, 0)),
        out_shape=jax.ShapeDtypeStruct((n, C), jnp.float32),
    )(x, W, b.reshape(1, C), gamma.reshape(1, C), beta.reshape(1, C))

    p3 = p.reshape(NB, SEG, 3)
    pt = jnp.transpose(p3, (0, 2, 1))
    sq = jnp.sum(p3 * p3, axis=2)[:, None, :]

    idx = pl.pallas_call(
        _knn_body,
        grid=(NB, SEG // QT),
        in_specs=[
            pl.BlockSpec((1, 3, SEG), lambda s, q: (s, 0, 0)),
            pl.BlockSpec((1, 1, SEG), lambda s, q: (s, 0, 0)),
            pl.BlockSpec((1, QT, 3), lambda s, q: (s, q, 0)),
        ],
        out_specs=pl.BlockSpec((1, QT, KMAX), lambda s, q: (s, q, 0)),
        out_shape=jax.ShapeDtypeStruct((NB, SEG, KMAX), jnp.int32),
        compiler_params=pltpu.CompilerParams(
            dimension_semantics=("arbitrary", "arbitrary"),
        ),
    )(pt, sq, p3)

    hp = jnp.pad(h, ((0, 0), (0, CP - C)))
    pooled = _pool(idx.reshape(N_TOT * KMAX), hp)
    out = jnp.concatenate([h, pooled], axis=1)
    return (p, out, o)


# speedup vs baseline: 1.5079x; 1.0272x over previous
import functools

import jax
import jax.numpy as jnp
from jax import lax
from jax.experimental import pallas as pl
from jax.experimental.pallas import tpu as pltpu
from jax.experimental.pallas import tpu_sc as plsc

C = 64
NB = 8
SEG = 4096
QT = 512
KMAX = 32
IMAX = 2147483647

N_TOT = NB * SEG
NW = 32
QPW = N_TOT // NW
BQ = 16
NBLK = QPW // BQ
CP = 128
NIDX = BQ * KMAX
NCH = NIDX // 128


def _h_body(x_ref, w_ref, b_ref, g_ref, be_ref, h_ref):
    h = jnp.dot(x_ref[...], w_ref[...], preferred_element_type=jnp.float32)
    h = (h + b_ref[...]) * g_ref[...] + be_ref[...]
    h_ref[...] = jnp.maximum(h, 0.0)


def _sortable(d):
    i = jax.lax.bitcast_convert_type(d, jnp.int32)
    return jnp.where(i >= 0, i, jnp.bitwise_xor(jnp.bitwise_not(i),
                                                -2147483648))


def _knn_body(seg0, pt_ref, sq_ref, pq_ref, idx_ref):
    pt = pt_ref[0]
    sq = sq_ref[0]
    pq = pq_ref[0]
    sq_q = jnp.sum(pq * pq, axis=1, keepdims=True)
    d = sq_q + sq - 2.0 * jnp.dot(pq, pt,
                                  preferred_element_type=jnp.float32)
    wk = d
    iota = lax.broadcasted_iota(jnp.int32, (QT, SEG), 1)
    base = (seg0 + pl.program_id(0)) * SEG
    idxs = []
    for _ in range(KMAX):
        am = jnp.argmin(wk, axis=1).astype(jnp.int32)[:, None]
        idxs.append(am + base)
        wk = jnp.where(iota == am, jnp.inf, wk)
    idx_ref[0] = jnp.concatenate(idxs, axis=1)


def _make_pool(nq):
    qpw = nq // NW
    nblk = qpw // BQ
    mesh = plsc.VectorSubcoreMesh(core_axis_name="c", subcore_axis_name="s")

    @functools.partial(
        pl.kernel, mesh=mesh,
        out_type=jax.ShapeDtypeStruct((nq, 3 * C), jnp.float32),
        scratch_types=[
            pltpu.VMEM((NIDX,), jnp.int32),
            pltpu.VMEM((NIDX, CP), jnp.float32),
            pltpu.VMEM((BQ, 3 * C), jnp.float32),
            pltpu.SemaphoreType.DMA,
        ],
    )
    def pool(idx_hbm, h_hbm, out_hbm, idx_v, rows_v, out_v, sem):
        wid = lax.axis_index("s") * 2 + lax.axis_index("c")
        base = wid * qpw

        def block(blk, carry):
            qb = base + blk * BQ
            pltpu.sync_copy(idx_hbm.at[pl.ds(qb * KMAX, NIDX)], idx_v)
            cps = [
                pltpu.async_copy(
                    h_hbm.at[idx_v.at[pl.ds(j * 128, 128)]],
                    rows_v.at[pl.ds(j * 128, 128), :], sem)
                for j in range(NCH)
            ]
            for cp in cps:
                cp.wait()

            def per_q(q, carry2):
                def accum(lo, hi, acc):
                    def rrow(r, a):
                        return tuple(
                            a[c] + rows_v[q * KMAX + r, pl.ds(c * 16, 16)]
                            for c in range(4))
                    return lax.fori_loop(lo, hi, rrow, acc)

                z = tuple(jnp.zeros((16,), jnp.float32) for _ in range(4))
                a8 = accum(0, 8, z)
                for c in range(4):
                    out_v[q, pl.ds(c * 16, 16)] = a8[c] * 0.125
                a16 = accum(8, 16, a8)
                for c in range(4):
                    out_v[q, pl.ds(C + c * 16, 16)] = a16[c] * 0.0625
                a32 = accum(16, 32, a16)
                for c in range(4):
                    out_v[q, pl.ds(2 * C + c * 16, 16)] = a32[c] * 0.03125
                return carry2

            lax.fori_loop(0, BQ, per_q, 0)
            pltpu.sync_copy(out_v, out_hbm.at[pl.ds(qb, BQ)])
            return carry

        lax.fori_loop(0, nblk, block, 0)

    return pool


NCHUNK = 4
SEG_PER_CHUNK = NB // NCHUNK
NQ_CHUNK = SEG_PER_CHUNK * SEG
_pool = _make_pool(NQ_CHUNK)


def kernel(p, x, o, W, b, gamma, beta):
    n = p.shape[0]
    h = pl.pallas_call(
        _h_body,
        grid=(n // 2048,),
        in_specs=[
            pl.BlockSpec((2048, C), lambda i: (i, 0)),
            pl.BlockSpec((C, C), lambda i: (0, 0)),
            pl.BlockSpec((1, C), lambda i: (0, 0)),
            pl.BlockSpec((1, C), lambda i: (0, 0)),
            pl.BlockSpec((1, C), lambda i: (0, 0)),
        ],
        out_specs=pl.BlockSpec((2048, C), lambda i: (i, 0)),
        out_shape=jax.ShapeDtypeStruct((n, C), jnp.float32),
    )(x, W, b.reshape(1, C), gamma.reshape(1, C), beta.reshape(1, C))

    p3 = p.reshape(NB, SEG, 3)
    pt = jnp.transpose(p3, (0, 2, 1))
    sq = jnp.sum(p3 * p3, axis=2)[:, None, :]

    hp = jnp.pad(h, ((0, 0), (0, CP - C)))

    pooled_chunks = []
    for ci in range(NCHUNK):
        s0 = ci * SEG_PER_CHUNK
        sl = slice(s0, s0 + SEG_PER_CHUNK)
        idx = pl.pallas_call(
            functools.partial(_knn_body, s0),
            grid=(SEG_PER_CHUNK, SEG // QT),
            in_specs=[
                pl.BlockSpec((1, 3, SEG), lambda s, q: (s, 0, 0)),
                pl.BlockSpec((1, 1, SEG), lambda s, q: (s, 0, 0)),
                pl.BlockSpec((1, QT, 3), lambda s, q: (s, q, 0)),
            ],
            out_specs=pl.BlockSpec((1, QT, KMAX), lambda s, q: (s, q, 0)),
            out_shape=jax.ShapeDtypeStruct((SEG_PER_CHUNK, SEG, KMAX),
                                           jnp.int32),
            compiler_params=pltpu.CompilerParams(
                dimension_semantics=("arbitrary", "arbitrary"),
            ),
        )(pt[sl], sq[sl], p3[sl])
        pooled_chunks.append(_pool(idx.reshape(NQ_CHUNK * KMAX), hp))

    out = jnp.concatenate([h] + [jnp.concatenate(pooled_chunks, axis=0)],
                          axis=1)
    return (p, out, o)


# speedup vs baseline: 1.5083x; 1.0003x over previous
import functools

import jax
import jax.numpy as jnp
from jax import lax
from jax.experimental import pallas as pl
from jax.experimental.pallas import tpu as pltpu
from jax.experimental.pallas import tpu_sc as plsc

C = 64
NB = 8
SEG = 4096
QT = 512
KMAX = 32

N_TOT = NB * SEG
NW = 32
QPW = N_TOT // NW
BQ = 16
NBLK = QPW // BQ
CP = 128
NIDX = BQ * KMAX
NCH = NIDX // 128


def _h_body(x_ref, w_ref, b_ref, g_ref, be_ref, h_ref):
    h = jnp.dot(x_ref[...], w_ref[...], preferred_element_type=jnp.float32)
    h = (h + b_ref[...]) * g_ref[...] + be_ref[...]
    h_ref[...] = jnp.maximum(h, 0.0)


def _knn_body(seg0, pt_ref, sq_ref, pq_ref, idx_ref):
    pt = pt_ref[0]
    sq = sq_ref[0]
    pq = pq_ref[0]
    sq_q = jnp.sum(pq * pq, axis=1, keepdims=True)
    d = sq_q + sq - 2.0 * jnp.dot(pq, pt,
                                  preferred_element_type=jnp.float32)
    wk = d
    iota = lax.broadcasted_iota(jnp.int32, (QT, SEG), 1)
    base = (seg0 + pl.program_id(0)) * SEG
    idxs = []
    for _ in range(KMAX):
        am = jnp.argmin(wk, axis=1).astype(jnp.int32)[:, None]
        idxs.append(am + base)
        wk = jnp.where(iota == am, jnp.inf, wk)
    idx_ref[0] = jnp.concatenate(idxs, axis=1)


def _make_pool(nq):
    qpw = nq // NW
    nblk = qpw // BQ
    mesh = plsc.VectorSubcoreMesh(core_axis_name="c", subcore_axis_name="s")

    @functools.partial(
        pl.kernel, mesh=mesh,
        out_type=jax.ShapeDtypeStruct((nq, 3 * C), jnp.float32),
        scratch_types=[
            pltpu.VMEM((NIDX,), jnp.int32),
            pltpu.VMEM((NIDX, CP), jnp.float32),
            pltpu.VMEM((BQ, 3 * C), jnp.float32),
            pltpu.SemaphoreType.DMA,
        ],
    )
    def pool(idx_hbm, h_hbm, out_hbm, idx_v, rows_v, out_v, sem):
        wid = lax.axis_index("s") * 2 + lax.axis_index("c")
        base = wid * qpw

        def block(blk, carry):
            qb = base + blk * BQ
            pltpu.sync_copy(idx_hbm.at[pl.ds(qb * KMAX, NIDX)], idx_v)
            cps = [
                pltpu.async_copy(
                    h_hbm.at[idx_v.at[pl.ds(j * 128, 128)]],
                    rows_v.at[pl.ds(j * 128, 128), :], sem)
                for j in range(NCH)
            ]
            for cp in cps:
                cp.wait()

            def per_q(q, carry2):
                def accum(lo, hi, acc):
                    def rrow(r, a):
                        return tuple(
                            a[c] + rows_v[q * KMAX + r, pl.ds(c * 16, 16)]
                            for c in range(4))
                    return lax.fori_loop(lo, hi, rrow, acc)

                z = tuple(jnp.zeros((16,), jnp.float32) for _ in range(4))
                a8 = accum(0, 8, z)
                for c in range(4):
                    out_v[q, pl.ds(c * 16, 16)] = a8[c] * 0.125
                a16 = accum(8, 16, a8)
                for c in range(4):
                    out_v[q, pl.ds(C + c * 16, 16)] = a16[c] * 0.0625
                a32 = accum(16, 32, a16)
                for c in range(4):
                    out_v[q, pl.ds(2 * C + c * 16, 16)] = a32[c] * 0.03125
                return carry2

            lax.fori_loop(0, BQ, per_q, 0)
            pltpu.sync_copy(out_v, out_hbm.at[pl.ds(qb, BQ)])
            return carry

        lax.fori_loop(0, nblk, block, 0)

    return pool


NCHUNK = 4
SEG_PER_CHUNK = NB // NCHUNK
NQ_CHUNK = SEG_PER_CHUNK * SEG
_pool = _make_pool(NQ_CHUNK)


def kernel(p, x, o, W, b, gamma, beta):
    n = p.shape[0]
    h = pl.pallas_call(
        _h_body,
        grid=(n // 2048,),
        in_specs=[
            pl.BlockSpec((2048, C), lambda i: (i, 0)),
            pl.BlockSpec((C, C), lambda i: (0, 0)),
            pl.BlockSpec((1, C), lambda i: (0, 0)),
            pl.BlockSpec((1, C), lambda i: (0, 0)),
            pl.BlockSpec((1, C), lambda i: (0, 0)),
        ],
        out_specs=pl.BlockSpec((2048, C), lambda i: (i, 0)),
        out_shape=jax.ShapeDtypeStruct((n, C), jnp.float32),
    )(x, W, b.reshape(1, C), gamma.reshape(1, C), beta.reshape(1, C))

    p3 = p.reshape(NB, SEG, 3)
    pt = jnp.transpose(p3, (0, 2, 1))
    sq = jnp.sum(p3 * p3, axis=2)[:, None, :]

    hp = jnp.pad(h, ((0, 0), (0, CP - C)))

    pooled_chunks = []
    for ci in range(NCHUNK):
        s0 = ci * SEG_PER_CHUNK
        sl = slice(s0, s0 + SEG_PER_CHUNK)
        idx = pl.pallas_call(
            functools.partial(_knn_body, s0),
            grid=(SEG_PER_CHUNK, SEG // QT),
            in_specs=[
                pl.BlockSpec((1, 3, SEG), lambda s, q: (s, 0, 0)),
                pl.BlockSpec((1, 1, SEG), lambda s, q: (s, 0, 0)),
                pl.BlockSpec((1, QT, 3), lambda s, q: (s, q, 0)),
            ],
            out_specs=pl.BlockSpec((1, QT, KMAX), lambda s, q: (s, q, 0)),
            out_shape=jax.ShapeDtypeStruct((SEG_PER_CHUNK, SEG, KMAX),
                                           jnp.int32),
            compiler_params=pltpu.CompilerParams(
                dimension_semantics=("arbitrary", "arbitrary"),
            ),
        )(pt[sl], sq[sl], p3[sl])
        pooled_chunks.append(_pool(idx.reshape(NQ_CHUNK * KMAX), hp))

    out = jnp.concatenate([h] + [jnp.concatenate(pooled_chunks, axis=0)],
                          axis=1)
    return (p, out, o)
